# baseline (speedup 1.0000x reference)
import functools

import jax
import jax.numpy as jnp
from jax import lax
from jax.experimental import pallas as pl
from jax.experimental.pallas import tpu as pltpu
from jax.experimental.pallas import tpu_sc as plsc

N_USERS = 25000
N_TOTAL = 50000
D = 64
DH = 32
NNZ = 800000
NS = 16
EB = 80
BLKS_PER_TILE = NNZ // NS // EB
NBUF = 5
NSC = 4
CH = 25
N_CHUNK = BLKS_PER_TILE // CH
OUTER = CH // NBUF
ROWS_PER_TILE = N_TOTAL // NS
N_RBLK = ROWS_PER_TILE // EB
R_TAIL = ROWS_PER_TILE - N_RBLK * EB

_mesh = plsc.VectorSubcoreMesh(core_axis_name="c", subcore_axis_name="s")

_f32 = jnp.float32
_half = jax.ShapeDtypeStruct((N_TOTAL, DH), _f32)
_emb = jax.ShapeDtypeStruct((N_USERS, D), _f32)


@functools.partial(
    pl.kernel,
    out_type=(_emb, _emb, _half, _half, _half, _half, _half, _half),
    mesh=_mesh,
    compiler_params=pltpu.CompilerParams(use_tc_tiling_on_sc=False),
    scratch_types=[
        pltpu.VMEM_SHARED((N_TOTAL, DH), _f32),
        pltpu.VMEM((CH, EB), jnp.int32),
        pltpu.VMEM((CH, EB), jnp.int32),
        pltpu.VMEM((CH * EB,), _f32),
        [pltpu.VMEM((EB, DH), _f32)] * NBUF,
        [pltpu.SemaphoreType.DMA] * NBUF,
        [pltpu.VMEM((EB, DH), _f32)] * NSC,
        [pltpu.SemaphoreType.DMA] * NSC,
    ],
)
def _lightgcn_sc(graph2, val_hbm, user_emb, item_emb,
                 users_out, items_out, x_lo, x_hi, l1_lo, l1_hi, l2_lo, l2_hi,
                 acc, col_v, row_v, val_v, ga, gs, sc, ss):
  cid = lax.axis_index("c")
  tid = lax.axis_index("s")
  rbase = tid * ROWS_PER_TILE
  c0 = cid * DH
  zeros16 = jnp.zeros((16,), _f32)

  def fill_zeros(buf):
    def body(r, _):
      buf[r, pl.ds(0, 16)] = zeros16
      buf[r, pl.ds(16, 16)] = zeros16
      return 0
    lax.fori_loop(0, EB, body, 0)

  def zero_acc_slice():
    def body(j, _):
      pltpu.sync_copy(ga[0], acc.at[pl.ds(rbase + j * EB, EB)])
      return 0
    lax.fori_loop(0, N_RBLK, body, 0)
    pltpu.sync_copy(ga[0].at[pl.ds(0, R_TAIL)],
                    acc.at[pl.ds(rbase + N_RBLK * EB, R_TAIL)])

  def build_x(x_hbm):
    @pl.when(tid < NS // 2)
    def _():
      pltpu.sync_copy(
          user_emb.at[pl.ds(rbase, ROWS_PER_TILE), pl.ds(c0, DH)],
          x_hbm.at[pl.ds(rbase, ROWS_PER_TILE)])
    @pl.when(tid >= NS // 2)
    def _():
      pltpu.sync_copy(
          item_emb.at[pl.ds(rbase - N_USERS, ROWS_PER_TILE), pl.ds(c0, DH)],
          x_hbm.at[pl.ds(rbase, ROWS_PER_TILE)])

  def scale(src, dst, vbase):
    def grp(g, _):
      vv = val_v[pl.ds(vbase + g * 16, 16)]
      e0 = g * 16
      for j in range(16):
        v = vv[j]
        dst[e0 + j, pl.ds(0, 16)] = src[e0 + j, pl.ds(0, 16)] * v
        dst[e0 + j, pl.ds(16, 16)] = src[e0 + j, pl.ds(16, 16)] * v
      return 0
    lax.fori_loop(0, EB // 16, grp, 0)

  def edge_pass(src_hbm):
    def chunk(ci, _):
      crow = tid * BLKS_PER_TILE + ci * CH
      pltpu.sync_copy(graph2.at[1, pl.ds(crow, CH)], col_v)
      pltpu.sync_copy(graph2.at[0, pl.ds(crow, CH)], row_v)
      pltpu.sync_copy(val_hbm.at[pl.ds(crow * EB, CH * EB)], val_v)
      for k in range(NBUF):
        pltpu.async_copy(src_hbm.at[col_v.at[k]], ga[k], gs[k])
      def outer(oi, _):
        for k in range(NBUF):
          s = k % NSC
          j = oi * NBUF + k
          pltpu.make_async_copy(src_hbm.at[col_v.at[j]], ga[k], gs[k]).wait()
          if k < NSC:
            @pl.when(jnp.logical_or(ci > 0, oi > 0))
            def _():
              pltpu.make_async_copy(sc[s], acc.at[row_v.at[j]], ss[s]).wait()
          else:
            pltpu.make_async_copy(sc[s], acc.at[row_v.at[j]], ss[s]).wait()
          scale(ga[k], sc[s], j * EB)
          pltpu.async_copy(sc[s], acc.at[row_v.at[j]], ss[s], add=True)
          @pl.when(oi < OUTER - 1)
          def _():
            pltpu.async_copy(src_hbm.at[col_v.at[j + NBUF]], ga[k], gs[k])
        return 0
      lax.fori_loop(0, OUTER, outer, 0)
      return 0
    lax.fori_loop(0, N_CHUNK, chunk, 0)
    for s in range(NSC):
      pltpu.make_async_copy(sc[s], acc.at[row_v.at[0]], ss[s]).wait()

  def writeback(dst_hbm):
    pltpu.sync_copy(acc.at[pl.ds(rbase, ROWS_PER_TILE)],
                    dst_hbm.at[pl.ds(rbase, ROWS_PER_TILE)])

  def combine(x_hbm, l1_hbm, l2_hbm):
    def do_rows(rs, nrows):
      cps = (
          pltpu.async_copy(x_hbm.at[pl.ds(rs, nrows)],
                           sc[0].at[pl.ds(0, nrows)], gs[0]),
          pltpu.async_copy(l1_hbm.at[pl.ds(rs, nrows)],
                           sc[1].at[pl.ds(0, nrows)], gs[1]),
          pltpu.async_copy(l2_hbm.at[pl.ds(rs, nrows)],
                           sc[2].at[pl.ds(0, nrows)], gs[2]),
          pltpu.async_copy(acc.at[pl.ds(rs, nrows)],
                           sc[3].at[pl.ds(0, nrows)], gs[3]),
      )
      for cp in cps:
        cp.wait()
      def body(r, _):
        for h in (0, 16):
          t = (sc[0][r, pl.ds(h, 16)] + sc[1][r, pl.ds(h, 16)]
               + sc[2][r, pl.ds(h, 16)] + sc[3][r, pl.ds(h, 16)])
          sc[0][r, pl.ds(h, 16)] = t * 0.25
        return 0
      lax.fori_loop(0, nrows, body, 0)
      @pl.when(tid < NS // 2)
      def _():
        pltpu.sync_copy(sc[0].at[pl.ds(0, nrows)],
                        users_out.at[pl.ds(rs, nrows), pl.ds(c0, DH)])
      @pl.when(tid >= NS // 2)
      def _():
        pltpu.sync_copy(sc[0].at[pl.ds(0, nrows)],
                        items_out.at[pl.ds(rs - N_USERS, nrows),
                                     pl.ds(c0, DH)])
    def chunkc(j, _):
      do_rows(rbase + j * EB, EB)
      return 0
    lax.fori_loop(0, N_RBLK, chunkc, 0)
    do_rows(rbase + N_RBLK * EB, R_TAIL)

  def propagate(x_hbm, l1_hbm, l2_hbm):
    build_x(x_hbm)
    fill_zeros(ga[0])
    zero_acc_slice()
    plsc.subcore_barrier()
    edge_pass(x_hbm)
    plsc.subcore_barrier()
    writeback(l1_hbm)
    fill_zeros(ga[0])
    zero_acc_slice()
    plsc.subcore_barrier()
    edge_pass(l1_hbm)
    plsc.subcore_barrier()
    writeback(l2_hbm)
    fill_zeros(ga[0])
    zero_acc_slice()
    plsc.subcore_barrier()
    edge_pass(l2_hbm)
    plsc.subcore_barrier()
    combine(x_hbm, l1_hbm, l2_hbm)

  @pl.when(cid == 0)
  def _():
    propagate(x_lo, l1_lo, l2_lo)

  @pl.when(cid == 1)
  def _():
    propagate(x_hi, l1_hi, l2_hi)


def kernel(user_emb, item_emb, graph_indices, graph_values):
  graph2 = graph_indices.reshape(2, NNZ // EB, EB)
  users, items = _lightgcn_sc(graph2, graph_values, user_emb, item_emb)[:2]
  return users, items


# speedup vs baseline: 1.2898x; 1.2898x over previous
import functools

import jax
import jax.numpy as jnp
from jax import lax
from jax.experimental import pallas as pl
from jax.experimental.pallas import tpu as pltpu
from jax.experimental.pallas import tpu_sc as plsc

N_USERS = 25000
N_TOTAL = 50000
D = 64
DH = 32
NNZ = 800000
NS = 16
EB = 80
NNZB = NNZ // EB
BLKS_PER_TILE = NNZB // NS
NBUF = 5
NSC = 3
CH = 25
N_CHUNK = BLKS_PER_TILE // CH
OUTER = CH // NBUF
ROWS_PER_TILE = N_TOTAL // NS
N_RBLK = ROWS_PER_TILE // EB
R_TAIL = ROWS_PER_TILE - N_RBLK * EB
BX = 40
N_BBLK = ROWS_PER_TILE // BX

_mesh = plsc.VectorSubcoreMesh(core_axis_name="c", subcore_axis_name="s")

_f32 = jnp.float32
_half = jax.ShapeDtypeStruct((N_TOTAL, DH), _f32)
_quart = jax.ShapeDtypeStruct((N_USERS, DH), _f32)


@functools.partial(
    pl.kernel,
    out_type=(_quart, _quart, _quart, _quart,
              _half, _half, _half, _half, _half, _half),
    mesh=_mesh,
    compiler_params=pltpu.CompilerParams(use_tc_tiling_on_sc=False),
    scratch_types=[
        pltpu.VMEM_SHARED((N_TOTAL, DH), _f32),
        pltpu.VMEM((CH, EB), jnp.int32),
        pltpu.VMEM((CH, EB), jnp.int32),
        pltpu.VMEM((CH * EB,), _f32),
        [pltpu.VMEM((EB, DH), _f32)] * NBUF,
        [pltpu.SemaphoreType.DMA] * NBUF,
        [pltpu.VMEM((EB, DH), _f32)] * NSC,
        [pltpu.SemaphoreType.DMA] * NSC,
        pltpu.VMEM((BX, D), _f32),
    ],
)
def _lightgcn_sc(g2, val_hbm, user_emb, item_emb,
                 users_lo, users_hi, items_lo, items_hi,
                 x_lo, x_hi, l1_lo, l1_hi, l2_lo, l2_hi,
                 acc, col_v, row_v, val_v, ga, gs, sc, ss, bx_v):
  cid = lax.axis_index("c")
  tid = lax.axis_index("s")
  rbase = tid * ROWS_PER_TILE
  c0 = cid * DH
  zeros16 = jnp.zeros((16,), _f32)

  def fill_zeros(buf):
    def body(r, _):
      buf[r, pl.ds(0, 16)] = zeros16
      buf[r, pl.ds(16, 16)] = zeros16
      return 0
    lax.fori_loop(0, EB, body, 0)

  def zero_acc_slice():
    def body(j, _):
      pltpu.sync_copy(ga[0], acc.at[pl.ds(rbase + j * EB, EB)])
      return 0
    lax.fori_loop(0, N_RBLK, body, 0)
    pltpu.sync_copy(ga[0].at[pl.ds(0, R_TAIL)],
                    acc.at[pl.ds(rbase + N_RBLK * EB, R_TAIL)])

  def build_x(x_hbm):
    def sel_rows(src_hbm, rs_loc, rs, nrows):
      pltpu.sync_copy(src_hbm.at[pl.ds(rs_loc, nrows)],
                      bx_v.at[pl.ds(0, nrows)])
      def body(r, _):
        sc[0][r, pl.ds(0, 16)] = bx_v[r, pl.ds(c0, 16)]
        sc[0][r, pl.ds(16, 16)] = bx_v[r, pl.ds(c0 + 16, 16)]
        return 0
      lax.fori_loop(0, nrows, body, 0)
      pltpu.sync_copy(sc[0].at[pl.ds(0, nrows)], x_hbm.at[pl.ds(rs, nrows)])
    def do(src_hbm, base_loc):
      def body(j, _):
        sel_rows(src_hbm, base_loc + j * BX, rbase + j * BX, BX)
        return 0
      lax.fori_loop(0, N_BBLK, body, 0)
      sel_rows(src_hbm, base_loc + N_BBLK * BX, rbase + N_BBLK * BX, R_TAIL)
    @pl.when(tid < NS // 2)
    def _():
      do(user_emb, rbase)
    @pl.when(tid >= NS // 2)
    def _():
      do(item_emb, rbase - N_USERS)

  def scale(src, dst, vbase):
    def grp(g, _):
      vv = val_v[pl.ds(vbase + g * 16, 16)]
      e0 = g * 16
      for j in range(16):
        v = vv[j]
        dst[e0 + j, pl.ds(0, 16)] = src[e0 + j, pl.ds(0, 16)] * v
        dst[e0 + j, pl.ds(16, 16)] = src[e0 + j, pl.ds(16, 16)] * v
      return 0
    lax.fori_loop(0, EB // 16, grp, 0)

  def edge_pass(src_hbm):
    def chunk(ci, _):
      crow = tid * BLKS_PER_TILE + ci * CH
      pltpu.sync_copy(g2.at[pl.ds(NNZB + crow, CH)], col_v)
      pltpu.sync_copy(g2.at[pl.ds(crow, CH)], row_v)
      pltpu.sync_copy(val_hbm.at[pl.ds(crow * EB, CH * EB)], val_v)
      for k in range(NBUF):
        pltpu.async_copy(src_hbm.at[col_v.at[k]], ga[k], gs[k])
      def outer(oi, _):
        for k in range(NBUF):
          s = k % NSC
          j = oi * NBUF + k
          pltpu.make_async_copy(src_hbm.at[col_v.at[j]], ga[k], gs[k]).wait()
          if k < NSC:
            @pl.when(jnp.logical_or(ci > 0, oi > 0))
            def _():
              pltpu.make_async_copy(sc[s], acc.at[row_v.at[j]], ss[s]).wait()
          else:
            pltpu.make_async_copy(sc[s], acc.at[row_v.at[j]], ss[s]).wait()
          scale(ga[k], sc[s], j * EB)
          pltpu.async_copy(sc[s], acc.at[row_v.at[j]], ss[s], add=True)
          @pl.when(oi < OUTER - 1)
          def _():
            pltpu.async_copy(src_hbm.at[col_v.at[j + NBUF]], ga[k], gs[k])
        return 0
      lax.fori_loop(0, OUTER, outer, 0)
      return 0
    lax.fori_loop(0, N_CHUNK, chunk, 0)
    for s in range(NSC):
      pltpu.make_async_copy(sc[s], acc.at[row_v.at[0]], ss[s]).wait()

  def writeback(dst_hbm):
    pltpu.sync_copy(acc.at[pl.ds(rbase, ROWS_PER_TILE)],
                    dst_hbm.at[pl.ds(rbase, ROWS_PER_TILE)])

  def combine(x_hbm, l1_hbm, l2_hbm, u_out, i_out):
    def do_rows(rs, nrows):
      cps = (
          pltpu.async_copy(x_hbm.at[pl.ds(rs, nrows)],
                           sc[0].at[pl.ds(0, nrows)], gs[0]),
          pltpu.async_copy(l1_hbm.at[pl.ds(rs, nrows)],
                           sc[1].at[pl.ds(0, nrows)], gs[1]),
          pltpu.async_copy(l2_hbm.at[pl.ds(rs, nrows)],
                           sc[2].at[pl.ds(0, nrows)], gs[2]),
          pltpu.async_copy(acc.at[pl.ds(rs, nrows)],
                           ga[0].at[pl.ds(0, nrows)], gs[3]),
      )
      for cp in cps:
        cp.wait()
      def body(r, _):
        for h in (0, 16):
          t = (sc[0][r, pl.ds(h, 16)] + sc[1][r, pl.ds(h, 16)]
               + sc[2][r, pl.ds(h, 16)] + ga[0][r, pl.ds(h, 16)])
          sc[0][r, pl.ds(h, 16)] = t * 0.25
        return 0
      lax.fori_loop(0, nrows, body, 0)
      @pl.when(tid < NS // 2)
      def _():
        pltpu.sync_copy(sc[0].at[pl.ds(0, nrows)],
                        u_out.at[pl.ds(rs, nrows)])
      @pl.when(tid >= NS // 2)
      def _():
        pltpu.sync_copy(sc[0].at[pl.ds(0, nrows)],
                        i_out.at[pl.ds(rs - N_USERS, nrows)])
    def chunkc(j, _):
      do_rows(rbase + j * EB, EB)
      return 0
    lax.fori_loop(0, N_RBLK, chunkc, 0)
    do_rows(rbase + N_RBLK * EB, R_TAIL)

  def propagate(x_hbm, l1_hbm, l2_hbm, u_out, i_out):
    build_x(x_hbm)
    fill_zeros(ga[0])
    zero_acc_slice()
    plsc.subcore_barrier()
    edge_pass(x_hbm)
    plsc.subcore_barrier()
    writeback(l1_hbm)
    fill_zeros(ga[0])
    zero_acc_slice()
    plsc.subcore_barrier()
    edge_pass(l1_hbm)
    plsc.subcore_barrier()
    writeback(l2_hbm)
    fill_zeros(ga[0])
    zero_acc_slice()
    plsc.subcore_barrier()
    edge_pass(l2_hbm)
    plsc.subcore_barrier()
    combine(x_hbm, l1_hbm, l2_hbm, u_out, i_out)

  @pl.when(cid == 0)
  def _():
    propagate(x_lo, l1_lo, l2_lo, users_lo, items_lo)

  @pl.when(cid == 1)
  def _():
    propagate(x_hi, l1_hi, l2_hi, users_hi, items_hi)


def kernel(user_emb, item_emb, graph_indices, graph_values):
  g2 = graph_indices.reshape(2 * NNZB, EB)
  outs = _lightgcn_sc(g2, graph_values, user_emb, item_emb)
  ul, uh, il, ih = outs[:4]
  users = jnp.concatenate([ul, uh], axis=1)
  items = jnp.concatenate([il, ih], axis=1)
  return users, items


# speedup vs baseline: 1.2915x; 1.0013x over previous
import functools

import jax
import jax.numpy as jnp
from jax import lax
from jax.experimental import pallas as pl
from jax.experimental.pallas import tpu as pltpu
from jax.experimental.pallas import tpu_sc as plsc

N_USERS = 25000
N_TOTAL = 50000
D = 64
DH = 32
NNZ = 800000
NS = 16
EB = 80
NNZB = NNZ // EB
BLKS_PER_TILE = NNZB // NS
NBUF = 5
NSC = 3
CH = 25
N_CHUNK = BLKS_PER_TILE // CH
OUTER = CH // NBUF
ROWS_PER_TILE = N_TOTAL // NS
N_RBLK = ROWS_PER_TILE // EB
R_TAIL = ROWS_PER_TILE - N_RBLK * EB
BX = 40
N_BBLK = ROWS_PER_TILE // BX

_mesh = plsc.VectorSubcoreMesh(core_axis_name="c", subcore_axis_name="s")

_f32 = jnp.float32
_half = jax.ShapeDtypeStruct((N_TOTAL, DH), _f32)
_quart = jax.ShapeDtypeStruct((N_USERS, DH), _f32)


@functools.partial(
    pl.kernel,
    out_type=(_quart, _quart, _quart, _quart),
    mesh=_mesh,
    compiler_params=pltpu.CompilerParams(use_tc_tiling_on_sc=False),
    scratch_types=[
        [pltpu.HBM((N_TOTAL, DH), _f32)] * 6,
        pltpu.VMEM_SHARED((N_TOTAL, DH), _f32),
        pltpu.VMEM((CH * EB,), jnp.int32),
        pltpu.VMEM((CH * EB,), jnp.int32),
        pltpu.VMEM((CH * EB,), _f32),
        [pltpu.VMEM((EB, DH), _f32)] * NBUF,
        [pltpu.SemaphoreType.DMA] * NBUF,
        [pltpu.VMEM((EB, DH), _f32)] * NSC,
        [pltpu.SemaphoreType.DMA] * NSC,
        pltpu.VMEM((BX, D), _f32),
    ],
)
def _lightgcn_sc(g_idx, val_hbm, user_emb, item_emb,
                 users_lo, users_hi, items_lo, items_hi,
                 hbm_s, acc, col_v, row_v, val_v, ga, gs, sc, ss, bx_v):
  x_lo, x_hi, l1_lo, l1_hi, l2_lo, l2_hi = hbm_s
  cid = lax.axis_index("c")
  tid = lax.axis_index("s")
  rbase = tid * ROWS_PER_TILE
  c0 = cid * DH
  zeros16 = jnp.zeros((16,), _f32)

  def fill_zeros(buf):
    def body(r, _):
      buf[r, pl.ds(0, 16)] = zeros16
      buf[r, pl.ds(16, 16)] = zeros16
      return 0
    lax.fori_loop(0, EB, body, 0)

  def zero_acc_slice():
    def body(j, _):
      pltpu.sync_copy(ga[0], acc.at[pl.ds(rbase + j * EB, EB)])
      return 0
    lax.fori_loop(0, N_RBLK, body, 0)
    pltpu.sync_copy(ga[0].at[pl.ds(0, R_TAIL)],
                    acc.at[pl.ds(rbase + N_RBLK * EB, R_TAIL)])

  def build_x(x_hbm):
    def sel_rows(src_hbm, rs_loc, rs, nrows):
      pltpu.sync_copy(src_hbm.at[pl.ds(rs_loc, nrows)],
                      bx_v.at[pl.ds(0, nrows)])
      def body(r, _):
        sc[0][r, pl.ds(0, 16)] = bx_v[r, pl.ds(c0, 16)]
        sc[0][r, pl.ds(16, 16)] = bx_v[r, pl.ds(c0 + 16, 16)]
        return 0
      lax.fori_loop(0, nrows, body, 0)
      pltpu.sync_copy(sc[0].at[pl.ds(0, nrows)], x_hbm.at[pl.ds(rs, nrows)])
    def do(src_hbm, base_loc):
      def body(j, _):
        sel_rows(src_hbm, base_loc + j * BX, rbase + j * BX, BX)
        return 0
      lax.fori_loop(0, N_BBLK, body, 0)
      sel_rows(src_hbm, base_loc + N_BBLK * BX, rbase + N_BBLK * BX, R_TAIL)
    @pl.when(tid < NS // 2)
    def _():
      do(user_emb, rbase)
    @pl.when(tid >= NS // 2)
    def _():
      do(item_emb, rbase - N_USERS)

  def scale(src, dst, vbase):
    def grp(g, _):
      vv = val_v[pl.ds(vbase + g * 16, 16)]
      e0 = g * 16
      for j in range(16):
        v = vv[j]
        dst[e0 + j, pl.ds(0, 16)] = src[e0 + j, pl.ds(0, 16)] * v
        dst[e0 + j, pl.ds(16, 16)] = src[e0 + j, pl.ds(16, 16)] * v
      return 0
    lax.fori_loop(0, EB // 16, grp, 0)

  def edge_pass(src_hbm):
    def chunk(ci, _):
      ebase = (tid * BLKS_PER_TILE + ci * CH) * EB
      pltpu.sync_copy(g_idx.at[1, pl.ds(ebase, CH * EB)], col_v)
      pltpu.sync_copy(g_idx.at[0, pl.ds(ebase, CH * EB)], row_v)
      pltpu.sync_copy(val_hbm.at[pl.ds(ebase, CH * EB)], val_v)
      for k in range(NBUF):
        pltpu.async_copy(src_hbm.at[col_v.at[pl.ds(k * EB, EB)]],
                         ga[k], gs[k])
      def outer(oi, _):
        for k in range(NBUF):
          s = k % NSC
          j = oi * NBUF + k
          cidx = col_v.at[pl.ds(j * EB, EB)]
          ridx = row_v.at[pl.ds(j * EB, EB)]
          pltpu.make_async_copy(src_hbm.at[cidx], ga[k], gs[k]).wait()
          if k < NSC:
            @pl.when(jnp.logical_or(ci > 0, oi > 0))
            def _():
              pltpu.make_async_copy(sc[s], acc.at[ridx], ss[s]).wait()
          else:
            pltpu.make_async_copy(sc[s], acc.at[ridx], ss[s]).wait()
          scale(ga[k], sc[s], j * EB)
          pltpu.async_copy(sc[s], acc.at[ridx], ss[s], add=True)
          @pl.when(oi < OUTER - 1)
          def _():
            pltpu.async_copy(src_hbm.at[col_v.at[pl.ds((j + NBUF) * EB, EB)]],
                             ga[k], gs[k])
        return 0
      lax.fori_loop(0, OUTER, outer, 0)
      return 0
    lax.fori_loop(0, N_CHUNK, chunk, 0)
    for s in range(NSC):
      pltpu.make_async_copy(sc[s], acc.at[row_v.at[pl.ds(0, EB)]],
                            ss[s]).wait()

  def writeback(dst_hbm):
    pltpu.sync_copy(acc.at[pl.ds(rbase, ROWS_PER_TILE)],
                    dst_hbm.at[pl.ds(rbase, ROWS_PER_TILE)])

  def combine(x_hbm, l1_hbm, l2_hbm, u_out, i_out):
    def do_rows(rs, nrows):
      cps = (
          pltpu.async_copy(x_hbm.at[pl.ds(rs, nrows)],
                           sc[0].at[pl.ds(0, nrows)], gs[0]),
          pltpu.async_copy(l1_hbm.at[pl.ds(rs, nrows)],
                           sc[1].at[pl.ds(0, nrows)], gs[1]),
          pltpu.async_copy(l2_hbm.at[pl.ds(rs, nrows)],
                           sc[2].at[pl.ds(0, nrows)], gs[2]),
          pltpu.async_copy(acc.at[pl.ds(rs, nrows)],
                           ga[0].at[pl.ds(0, nrows)], gs[3]),
      )
      for cp in cps:
        cp.wait()
      def body(r, _):
        for h in (0, 16):
          t = (sc[0][r, pl.ds(h, 16)] + sc[1][r, pl.ds(h, 16)]
               + sc[2][r, pl.ds(h, 16)] + ga[0][r, pl.ds(h, 16)])
          sc[0][r, pl.ds(h, 16)] = t * 0.25
        return 0
      lax.fori_loop(0, nrows, body, 0)
      @pl.when(tid < NS // 2)
      def _():
        pltpu.sync_copy(sc[0].at[pl.ds(0, nrows)],
                        u_out.at[pl.ds(rs, nrows)])
      @pl.when(tid >= NS // 2)
      def _():
        pltpu.sync_copy(sc[0].at[pl.ds(0, nrows)],
                        i_out.at[pl.ds(rs - N_USERS, nrows)])
    def chunkc(j, _):
      do_rows(rbase + j * EB, EB)
      return 0
    lax.fori_loop(0, N_RBLK, chunkc, 0)
    do_rows(rbase + N_RBLK * EB, R_TAIL)

  def propagate(x_hbm, l1_hbm, l2_hbm, u_out, i_out):
    build_x(x_hbm)
    fill_zeros(ga[0])
    zero_acc_slice()
    plsc.subcore_barrier()
    edge_pass(x_hbm)
    plsc.subcore_barrier()
    writeback(l1_hbm)
    fill_zeros(ga[0])
    zero_acc_slice()
    plsc.subcore_barrier()
    edge_pass(l1_hbm)
    plsc.subcore_barrier()
    writeback(l2_hbm)
    fill_zeros(ga[0])
    zero_acc_slice()
    plsc.subcore_barrier()
    edge_pass(l2_hbm)
    plsc.subcore_barrier()
    combine(x_hbm, l1_hbm, l2_hbm, u_out, i_out)

  @pl.when(cid == 0)
  def _():
    propagate(x_lo, l1_lo, l2_lo, users_lo, items_lo)

  @pl.when(cid == 1)
  def _():
    propagate(x_hi, l1_hi, l2_hi, users_hi, items_hi)


def kernel(user_emb, item_emb, graph_indices, graph_values):
  ul, uh, il, ih = _lightgcn_sc(graph_indices, graph_values,
                                user_emb, item_emb)
  users = jnp.concatenate([ul, uh], axis=1)
  items = jnp.concatenate([il, ih], axis=1)
  return users, items
